# baseline (speedup 1.0000x reference)
import jax
import jax.numpy as jnp
from jax import lax
from jax.experimental import pallas as pl
from jax.experimental.pallas import tpu as pltpu
from jax.experimental.pallas import tpu_sc as plsc

_IMG = 512
_PATCH = 16
_G = 32
_N = _G * _G
_B = 8
_C = 96
_CB = 16


def _pool_body(x_ref, o_ref):
    c = pl.program_id(1)
    xb = x_ref[0]
    y = xb[0]
    for k in range(1, _CB):
        y = y + xb[k]
    g_i = lax.broadcasted_iota(jnp.int32, (_G, _IMG), 0)
    h_i = lax.broadcasted_iota(jnp.int32, (_G, _IMG), 1)
    at = (h_i // _PATCH == g_i).astype(jnp.float32)
    w_i = lax.broadcasted_iota(jnp.int32, (_IMG, _G), 0)
    gg_i = lax.broadcasted_iota(jnp.int32, (_IMG, _G), 1)
    a = (w_i // _PATCH == gg_i).astype(jnp.float32)
    r = lax.dot(at, y, precision=lax.Precision.HIGHEST,
                preferred_element_type=jnp.float32)
    p = lax.dot(r, a, precision=lax.Precision.HIGHEST,
                preferred_element_type=jnp.float32)

    @pl.when(c == 0)
    def _():
        o_ref[0] = p

    @pl.when(c != 0)
    def _():
        o_ref[0] += p


def _pool(x):
    return pl.pallas_call(
        _pool_body,
        grid=(_B, _C // _CB),
        in_specs=[pl.BlockSpec((1, _CB, _IMG, _IMG), lambda b, c: (b, c, 0, 0))],
        out_specs=pl.BlockSpec((1, _G, _G), lambda b, c: (b, 0, 0)),
        out_shape=jax.ShapeDtypeStruct((_B, _G, _G), jnp.float32),
        compiler_params=pltpu.CompilerParams(
            dimension_semantics=("parallel", "arbitrary")),
    )(x)


_Q = _N // 4


def _rank_body(sums_hbm, avgx_hbm, rank_hbm, vals_v, avgx_v, rank_v):
    cid = lax.axis_index("c")
    sid = lax.axis_index("s")
    wid = cid * 16 + sid
    b = wid // 4
    q = wid % 4

    pltpu.sync_copy(sums_hbm.at[b], vals_v)

    v0 = vals_v[pl.ds(0, 16)]

    def _mm(u, carry):
        mn, mx = carry
        v = vals_v[pl.ds(u * 16, 16)]
        return jnp.minimum(mn, v), jnp.maximum(mx, v)

    mnv, mxv = lax.fori_loop(1, _N // 16, _mm, (v0, v0))
    mn = mnv[0]
    mx = mxv[0]
    for m in range(1, 16):
        mn = jnp.minimum(mn, mnv[m])
        mx = jnp.maximum(mx, mxv[m])
    rng = mx - mn

    def _nrm(u, carry):
        v = vals_v[pl.ds((q * 16 + u) * 16, 16)]
        avgx_v[pl.ds(u * 16, 16)] = (v - mn) / rng
        return carry

    lax.fori_loop(0, _Q // 16, _nrm, 0)

    def _rank_chunk(tl, carry):
        t = q * 16 + tl
        jb = t * 16
        vj = vals_v[pl.ds(jb, 16)]
        gj = jb + lax.iota(jnp.int32, 16)
        acc = jnp.zeros((16,), jnp.int32)

        def _lo(u, acc):
            vi = vals_v[pl.ds(u * 16, 16)]
            for m in range(16):
                si = vi[m]
                acc = acc + jnp.where(vj >= si, 1, 0)
            return acc

        acc = lax.fori_loop(0, t, _lo, acc)

        for m in range(16):
            i = jb + m
            si = vj[m]
            tie = jnp.where(gj > i, 1, 0)
            acc = acc + jnp.where(vj > si, 1,
                                  jnp.where(vj == si, tie, 0))

        def _hi(u, acc):
            vi = vals_v[pl.ds(u * 16, 16)]
            for m in range(16):
                si = vi[m]
                acc = acc + jnp.where(vj > si, 1, 0)
            return acc

        acc = lax.fori_loop(t + 1, _N // 16, _hi, acc)

        rank_v[pl.ds(tl * 16, 16)] = acc
        return carry

    lax.fori_loop(0, _Q // 16, _rank_chunk, 0)

    pltpu.sync_copy(avgx_v, avgx_hbm.at[wid])
    pltpu.sync_copy(rank_v, rank_hbm.at[wid])


def _rank(sums):
    return pl.kernel(
        _rank_body,
        out_type=(jax.ShapeDtypeStruct((4 * _B, _Q), jnp.float32),
                  jax.ShapeDtypeStruct((4 * _B, _Q), jnp.int32)),
        mesh=plsc.VectorSubcoreMesh(core_axis_name="c", subcore_axis_name="s"),
        scratch_types=[pltpu.VMEM((_N,), jnp.float32),
                       pltpu.VMEM((_Q,), jnp.float32),
                       pltpu.VMEM((_Q,), jnp.int32)],
    )(sums)


def _ids_body(rank_hbm, ids_hbm, rk_v, ids_v):
    cid = lax.axis_index("c")
    sid = lax.axis_index("s")
    wid = cid * 16 + sid
    b = wid // 4
    q = wid % 4

    for r in range(4):
        pltpu.sync_copy(rank_hbm.at[4 * b + r], rk_v.at[pl.ds(_Q * r, _Q)])

    kbase = q * _Q

    def _ids_chunk(kl, carry):
        kvec = kbase + kl * 16 + lax.iota(jnp.int32, 16)
        acc = jnp.zeros((16,), jnp.int32)

        def _scan(u, acc):
            rvec = rk_v[pl.ds(u * 16, 16)]
            for m in range(16):
                rj = rvec[m]
                jglob = u * 16 + m
                acc = acc + jnp.where(kvec == rj, jglob, 0)
            return acc

        acc = lax.fori_loop(0, _N // 16, _scan, acc)
        ids_v[pl.ds(kl * 16, 16)] = acc
        return carry

    lax.fori_loop(0, _Q // 16, _ids_chunk, 0)

    pltpu.sync_copy(ids_v, ids_hbm.at[wid])


def _ids(rank):
    return pl.kernel(
        _ids_body,
        out_type=jax.ShapeDtypeStruct((4 * _B, _Q), jnp.int32),
        mesh=plsc.VectorSubcoreMesh(core_axis_name="c", subcore_axis_name="s"),
        scratch_types=[pltpu.VMEM((_N,), jnp.int32),
                       pltpu.VMEM((_Q,), jnp.int32)],
    )(rank)


def kernel(x):
    sums = _pool(x)
    avg_x, rank = _rank(sums.reshape(_B, _N))
    ids = _ids(rank).reshape(_B, _N)
    return avg_x.reshape(_B, _N), ids


# speedup vs baseline: 1.0123x; 1.0123x over previous
import jax
import jax.numpy as jnp
from jax import lax
from jax.experimental import pallas as pl
from jax.experimental.pallas import tpu as pltpu
from jax.experimental.pallas import tpu_sc as plsc

_IMG = 512
_PATCH = 16
_G = 32
_N = _G * _G
_B = 8
_C = 96
_CB = 16


def _pool_body(x_ref, o_ref):
    c = pl.program_id(1)
    xb = x_ref[0]
    y = xb[0]
    for k in range(1, _CB):
        y = y + xb[k]
    g_i = lax.broadcasted_iota(jnp.int32, (_G, _IMG), 0)
    h_i = lax.broadcasted_iota(jnp.int32, (_G, _IMG), 1)
    at = (h_i // _PATCH == g_i).astype(jnp.float32)
    w_i = lax.broadcasted_iota(jnp.int32, (_IMG, _G), 0)
    gg_i = lax.broadcasted_iota(jnp.int32, (_IMG, _G), 1)
    a = (w_i // _PATCH == gg_i).astype(jnp.float32)
    r = lax.dot(at, y, precision=lax.Precision.HIGHEST,
                preferred_element_type=jnp.float32)
    p = lax.dot(r, a, precision=lax.Precision.HIGHEST,
                preferred_element_type=jnp.float32)

    @pl.when(c == 0)
    def _():
        o_ref[0] = p

    @pl.when(c != 0)
    def _():
        o_ref[0] += p


def _pool(x):
    return pl.pallas_call(
        _pool_body,
        grid=(_B, _C // _CB),
        in_specs=[pl.BlockSpec((1, _CB, _IMG, _IMG), lambda b, c: (b, c, 0, 0))],
        out_specs=pl.BlockSpec((1, _G, _G), lambda b, c: (b, 0, 0)),
        out_shape=jax.ShapeDtypeStruct((_B, _G, _G), jnp.float32),
        compiler_params=pltpu.CompilerParams(
            dimension_semantics=("parallel", "arbitrary")),
    )(x)


_Q = _N // 4


def _rank_body(sums_hbm, avgx_hbm, ids_hbm, vals_v, avgx_v, rank_v, rank_sh,
               rk_v, ids_v):
    cid = lax.axis_index("c")
    sid = lax.axis_index("s")
    wid = cid * 16 + sid
    b = wid // 4
    q = wid % 4

    pltpu.sync_copy(sums_hbm.at[b], vals_v)

    v0 = vals_v[pl.ds(0, 16)]

    def _mm(u, carry):
        mn, mx = carry
        v = vals_v[pl.ds(u * 16, 16)]
        return jnp.minimum(mn, v), jnp.maximum(mx, v)

    mnv, mxv = lax.fori_loop(1, _N // 16, _mm, (v0, v0))
    mn = mnv[0]
    mx = mxv[0]
    for m in range(1, 16):
        mn = jnp.minimum(mn, mnv[m])
        mx = jnp.maximum(mx, mxv[m])
    rng = mx - mn

    def _nrm(u, carry):
        v = vals_v[pl.ds((q * 16 + u) * 16, 16)]
        avgx_v[pl.ds(u * 16, 16)] = (v - mn) / rng
        return carry

    lax.fori_loop(0, _Q // 16, _nrm, 0)

    def _rank_chunk(tl, carry):
        t = q * 16 + tl
        jb = t * 16
        vj = vals_v[pl.ds(jb, 16)]
        gj = jb + lax.iota(jnp.int32, 16)
        acc = jnp.zeros((16,), jnp.int32)

        def _lo(u, acc):
            vi = vals_v[pl.ds(u * 16, 16)]
            for m in range(16):
                si = vi[m]
                acc = acc + jnp.where(vj >= si, 1, 0)
            return acc

        acc = lax.fori_loop(0, t, _lo, acc)

        for m in range(16):
            i = jb + m
            si = vj[m]
            tie = jnp.where(gj > i, 1, 0)
            acc = acc + jnp.where(vj > si, 1,
                                  jnp.where(vj == si, tie, 0))

        def _hi(u, acc):
            vi = vals_v[pl.ds(u * 16, 16)]
            for m in range(16):
                si = vi[m]
                acc = acc + jnp.where(vj > si, 1, 0)
            return acc

        acc = lax.fori_loop(t + 1, _N // 16, _hi, acc)

        rank_v[pl.ds(tl * 16, 16)] = acc
        return carry

    lax.fori_loop(0, _Q // 16, _rank_chunk, 0)

    pltpu.sync_copy(avgx_v, avgx_hbm.at[wid])

    pltpu.sync_copy(rank_v, rank_sh.at[sid])
    plsc.subcore_barrier()

    for r in range(4):
        pltpu.sync_copy(rank_sh.at[(b % 4) * 4 + r], rk_v.at[pl.ds(_Q * r, _Q)])

    kbase = q * _Q

    def _ids_chunk(kl, carry):
        kvec = kbase + kl * 16 + lax.iota(jnp.int32, 16)
        acc = jnp.zeros((16,), jnp.int32)

        def _scan(u, acc):
            rvec = rk_v[pl.ds(u * 16, 16)]
            for m in range(16):
                rj = rvec[m]
                jglob = u * 16 + m
                acc = acc + jnp.where(kvec == rj, jglob, 0)
            return acc

        acc = lax.fori_loop(0, _N // 16, _scan, acc)
        ids_v[pl.ds(kl * 16, 16)] = acc
        return carry

    lax.fori_loop(0, _Q // 16, _ids_chunk, 0)

    pltpu.sync_copy(ids_v, ids_hbm.at[wid])


def _rank(sums):
    return pl.kernel(
        _rank_body,
        out_type=(jax.ShapeDtypeStruct((4 * _B, _Q), jnp.float32),
                  jax.ShapeDtypeStruct((4 * _B, _Q), jnp.int32)),
        mesh=plsc.VectorSubcoreMesh(core_axis_name="c", subcore_axis_name="s"),
        scratch_types=[pltpu.VMEM((_N,), jnp.float32),
                       pltpu.VMEM((_Q,), jnp.float32),
                       pltpu.VMEM((_Q,), jnp.int32),
                       pltpu.VMEM_SHARED((16, _Q), jnp.int32),
                       pltpu.VMEM((_N,), jnp.int32),
                       pltpu.VMEM((_Q,), jnp.int32)],
    )(sums)


def kernel(x):
    sums = _pool(x)
    avg_x, ids = _rank(sums.reshape(_B, _N))
    return avg_x.reshape(_B, _N), ids.reshape(_B, _N)
